# Initial kernel scaffold; baseline (speedup 1.0000x reference)
import jax
import jax.numpy as jnp
from jax.experimental import pallas as pl


def kernel(logprobs, beam_logprobs_sum, beam_seq, beam_seq_logprobs, state):
    raise NotImplementedError("write your pallas kernel here")


# speedup vs baseline: 19.8955x; 19.8955x over previous
import jax
import jax.numpy as jnp
from jax import lax
from jax.experimental import pallas as pl
from jax.experimental.pallas import tpu as pltpu


def _topk_body(lp_ref, bias_ref, seq_ref, seq_out_ref, sum_out_ref, src_out_ref):
    nb = lp_ref.shape[1]
    v = lp_ref.shape[2]
    t = seq_ref.shape[2]
    b = pl.program_id(0)
    x = lp_ref[0] + bias_ref[0][:, 0:1]
    rowi = lax.broadcasted_iota(jnp.int32, (nb, v), 0)
    coli = lax.broadcasted_iota(jnp.int32, (nb, v), 1)
    flat = rowi * v + coli
    vals = jnp.zeros((1, nb), jnp.float32)
    srcs = jnp.zeros((1, nb), jnp.int32)
    selc = jnp.zeros((nb, 1), jnp.int32)
    prefix = jnp.zeros((nb, t), jnp.int32)
    li = lax.broadcasted_iota(jnp.int32, (1, nb), 1)
    ri = lax.broadcasted_iota(jnp.int32, (nb, 1), 0)
    rt = lax.broadcasted_iota(jnp.int32, (nb, t), 0)
    for k in range(nb):
        m = jnp.max(x)
        idx = jnp.min(jnp.where(x >= m, flat, jnp.int32(nb * v)))
        bix = idx // v
        sel = idx - bix * v
        seq_row = seq_ref[0, pl.ds(bix, 1), :]
        vals = jnp.where(li == k, m, vals)
        srcs = jnp.where(li == k, bix, srcs)
        selc = jnp.where(ri == k, sel, selc)
        prefix = jnp.where(rt == k, seq_row, prefix)
        x = jnp.where(flat == idx, -jnp.inf, x)
    sum_out_ref[0] = vals
    src_out_ref[0] = srcs + b * nb
    seq_out_ref[0] = jnp.concatenate([prefix, selc], axis=1).astype(seq_out_ref.dtype)


def _gather_body(src_ref, slp_ref, lp_ref, st_ref, oslp_ref, ost_ref):
    t = slp_ref.shape[1]
    oslp_ref[0, 0:t, :] = slp_ref[0]
    oslp_ref[0, t:t + 1, :] = lp_ref[0]
    ost_ref[...] = st_ref[...]


def kernel(logprobs, beam_logprobs_sum, beam_seq, beam_seq_logprobs, state):
    B, BD = beam_logprobs_sum.shape
    V = logprobs.shape[-1]
    T = beam_seq.shape[-1]
    L, R, D = state.shape

    lp3 = logprobs.reshape(B, BD, V)
    bias = jnp.broadcast_to(beam_logprobs_sum[:, :, None], (B, BD, 128))

    seq_out, sums, srcs = pl.pallas_call(
        _topk_body,
        grid=(B,),
        in_specs=[
            pl.BlockSpec((1, BD, V), lambda b: (b, 0, 0)),
            pl.BlockSpec((1, BD, 128), lambda b: (b, 0, 0)),
            pl.BlockSpec((1, BD, T), lambda b: (b, 0, 0)),
        ],
        out_specs=[
            pl.BlockSpec((1, BD, T + 1), lambda b: (b, 0, 0)),
            pl.BlockSpec((1, 1, BD), lambda b: (b, 0, 0)),
            pl.BlockSpec((1, 1, BD), lambda b: (b, 0, 0)),
        ],
        out_shape=[
            jax.ShapeDtypeStruct((B, BD, T + 1), beam_seq.dtype),
            jax.ShapeDtypeStruct((B, 1, BD), jnp.float32),
            jax.ShapeDtypeStruct((B, 1, BD), jnp.int32),
        ],
    )(lp3, bias, beam_seq)

    srcflat = srcs.reshape(-1)
    slp4 = beam_seq_logprobs.reshape(B * BD, T, V)
    lp3r = logprobs.reshape(B * BD, 1, V)
    st4 = state.reshape(L, R, 1, D)

    grid_spec = pltpu.PrefetchScalarGridSpec(
        num_scalar_prefetch=1,
        grid=(B * BD,),
        in_specs=[
            pl.BlockSpec((1, T, V), lambda i, s: (s[i], 0, 0)),
            pl.BlockSpec((1, 1, V), lambda i, s: (s[i], 0, 0)),
            pl.BlockSpec((L, 1, 1, D), lambda i, s: (0, s[i], 0, 0)),
        ],
        out_specs=[
            pl.BlockSpec((1, T + 1, V), lambda i, s: (i, 0, 0)),
            pl.BlockSpec((L, 1, 1, D), lambda i, s: (0, i, 0, 0)),
        ],
    )
    oslp, ost = pl.pallas_call(
        _gather_body,
        grid_spec=grid_spec,
        out_shape=[
            jax.ShapeDtypeStruct((B * BD, T + 1, V), jnp.float32),
            jax.ShapeDtypeStruct((L, R, 1, D), jnp.float32),
        ],
    )(srcflat, slp4, lp3r, st4)

    return (seq_out,
            oslp.reshape(B, BD, T + 1, V),
            sums.reshape(B, BD),
            ost.reshape(L, R, D))
